# baseline (speedup 1.0000x reference)
import functools

import jax
import jax.numpy as jnp
from jax import lax
from jax.experimental import pallas as pl
from jax.experimental.pallas import tpu as pltpu
from jax.experimental.pallas import tpu_sc as plsc

_N_EMB = 8192
_EMB_DIM = 32
_BETA = 0.25
_ROWS = 16384
_R = 256
_CHUNK = 1024
_HALF = _N_EMB // 2
_LANES = 128

_NW = 32
_IDX_CHUNK = 128
_B_PER_W = _ROWS // _NW
_CH_PER_W = _B_PER_W // _IDX_CHUNK


def _assign_body(zsq_ref, esq_ref, z_ref, embr_ref, idx_ref, dsum_ref):
    zb = z_ref[0].astype(jnp.bfloat16)
    zsq = zsq_ref[...]
    lane = lax.broadcasted_iota(
        jnp.int32, (_R, _LANES), 1).astype(jnp.float32)
    halves = []
    for h in range(2):
        acc_v = jnp.full((_R, _LANES), jnp.inf, dtype=jnp.float32)
        acc_i = jnp.zeros((_R, _LANES), dtype=jnp.float32)
        for c in range(_HALF // _CHUNK):
            c0 = h * _HALF + c * _CHUNK
            p2 = lax.dot_general(
                zb, embr_ref[c0:c0 + _CHUNK, :],
                dimension_numbers=(((0,), (1,)), ((), ())),
                preferred_element_type=jnp.float32)
            for s in range(_CHUNK // _LANES):
                s0 = s * _LANES
                d_s = (zsq + esq_ref[:, c0 + s0:c0 + s0 + _LANES]) \
                    + p2[:, s0:s0 + _LANES]
                take = d_s < acc_v
                acc_v = jnp.minimum(acc_v, d_s)
                acc_i = jnp.where(take, float((c0 + s0) // _LANES), acc_i)
        m = jnp.min(acc_v, axis=1, keepdims=True)
        gidx = acc_i * float(_LANES) + lane
        li = jnp.min(jnp.where(acc_v == m, gidx, float(_N_EMB)),
                     axis=1, keepdims=True)
        halves.append((m, li))
    (va, ia), (vb, ib) = halves
    va_b = va.astype(jnp.bfloat16).astype(jnp.float32)
    pick_b = vb < va_b
    idx_ref[...] = jnp.where(pick_b, ib, ia).astype(jnp.int32)
    vsel = jnp.where(pick_b, vb, va)

    @pl.when(pl.program_id(0) == 0)
    def _init():
        dsum_ref[...] = jnp.zeros_like(dsum_ref)

    dsum_ref[...] += jnp.sum(vsel)


def _assign(z, zsq, esq, embr_bf16):
    grid = (_ROWS // _R,)
    blocks_per_batch = 2048 // _R
    return pl.pallas_call(
        _assign_body,
        grid=grid,
        in_specs=[
            pl.BlockSpec((_R, 1), lambda i: (i, 0)),
            pl.BlockSpec((1, _N_EMB), lambda i: (0, 0)),
            pl.BlockSpec((1, _EMB_DIM, _R),
                         lambda i: (i // blocks_per_batch, 0,
                                    i % blocks_per_batch)),
            pl.BlockSpec((_N_EMB, _EMB_DIM), lambda i: (0, 0)),
        ],
        out_specs=[
            pl.BlockSpec((_R, 1), lambda i: (i, 0)),
            pl.BlockSpec((1, 1), lambda i: (0, 0)),
        ],
        out_shape=[
            jax.ShapeDtypeStruct((_ROWS, 1), jnp.int32),
            jax.ShapeDtypeStruct((1, 1), jnp.float32),
        ],
    )(zsq, esq, z, embr_bf16)


def _sc_gather_body(table_hbm, idx_hbm, out_hbm, idx_v, rows_v, sem):
    wid = lax.axis_index("s") * 2 + lax.axis_index("c")
    pltpu.sync_copy(idx_hbm.at[pl.ds(wid * _CH_PER_W, _CH_PER_W)], idx_v)
    copies = []
    for j in range(_CH_PER_W):
        copies.append(pltpu.async_copy(
            table_hbm.at[idx_v.at[j]],
            rows_v.at[pl.ds(j * _IDX_CHUNK, _IDX_CHUNK)],
            sem))
    for cp in copies:
        cp.wait()
    pltpu.sync_copy(rows_v, out_hbm.at[pl.ds(wid * _B_PER_W, _B_PER_W)])


def _sc_gather(emb_weight, idx_grid):
    return pl.kernel(
        _sc_gather_body,
        out_type=jax.ShapeDtypeStruct((_ROWS, _EMB_DIM), jnp.float32),
        mesh=plsc.VectorSubcoreMesh(core_axis_name="c", subcore_axis_name="s"),
        scratch_types=[
            pltpu.VMEM((_CH_PER_W, _IDX_CHUNK), jnp.int32),
            pltpu.VMEM((_B_PER_W, _EMB_DIM), jnp.float32),
            pltpu.SemaphoreType.DMA,
        ],
        compiler_params=pltpu.CompilerParams(use_tc_tiling_on_sc=False),
    )(emb_weight, idx_grid)


def kernel(z, emb_weight):
    zsq = jnp.sum(z ** 2, axis=1).reshape(_ROWS, 1)
    esq = jnp.sum(emb_weight ** 2, axis=1).reshape(1, _N_EMB)
    embr = (-2.0 * emb_weight).astype(jnp.bfloat16)

    idx2, dsum = _assign(z, zsq, esq, embr)
    encoding_indices = idx2.reshape(_ROWS)

    idx_grid = idx2.reshape(_NW * _CH_PER_W, _IDX_CHUNK)
    z_q_flat = _sc_gather(emb_weight, idx_grid)

    z_q_t = jnp.transpose(z_q_flat.reshape(8, 2048, _EMB_DIM), (0, 2, 1))
    m = dsum[0, 0] / (_ROWS * _EMB_DIM)
    loss = _BETA * m + m
    z_q_out = z + (z_q_t - z)
    return (z_q_out, loss, encoding_indices)


# speedup vs baseline: 1.1035x; 1.1035x over previous
import functools

import jax
import jax.numpy as jnp
from jax import lax
from jax.experimental import pallas as pl
from jax.experimental.pallas import tpu as pltpu
from jax.experimental.pallas import tpu_sc as plsc

_N_EMB = 8192
_EMB_DIM = 32
_BETA = 0.25
_ROWS = 16384
_R = 512
_CHUNK = 1024
_HALF = _N_EMB // 2
_LANES = 128

_NW = 32
_IDX_CHUNK = 128
_B_PER_W = _ROWS // _NW
_CH_PER_W = _B_PER_W // _IDX_CHUNK


def _assign_body(zsq_ref, esq_ref, z_ref, embr_ref, idx_ref, dsum_ref):
    zb = z_ref[0].astype(jnp.bfloat16)
    zsq = zsq_ref[...]
    lane = lax.broadcasted_iota(
        jnp.int32, (_R, _LANES), 1).astype(jnp.float32)
    halves = []
    for h in range(2):
        acc_v = jnp.full((_R, _LANES), jnp.inf, dtype=jnp.float32)
        acc_i = jnp.zeros((_R, _LANES), dtype=jnp.float32)
        for c in range(_HALF // _CHUNK):
            c0 = h * _HALF + c * _CHUNK
            p2 = lax.dot_general(
                zb, embr_ref[c0:c0 + _CHUNK, :],
                dimension_numbers=(((0,), (1,)), ((), ())),
                preferred_element_type=jnp.float32)
            for s in range(_CHUNK // _LANES):
                s0 = s * _LANES
                d_s = (zsq + esq_ref[:, c0 + s0:c0 + s0 + _LANES]) \
                    + p2[:, s0:s0 + _LANES]
                take = d_s < acc_v
                acc_v = jnp.minimum(acc_v, d_s)
                acc_i = jnp.where(take, float((c0 + s0) // _LANES), acc_i)
        m = jnp.min(acc_v, axis=1, keepdims=True)
        gidx = acc_i * float(_LANES) + lane
        li = jnp.min(jnp.where(acc_v == m, gidx, float(_N_EMB)),
                     axis=1, keepdims=True)
        halves.append((m, li))
    (va, ia), (vb, ib) = halves
    va_b = va.astype(jnp.bfloat16).astype(jnp.float32)
    pick_b = vb < va_b
    idx_ref[...] = jnp.where(pick_b, ib, ia).astype(jnp.int32)
    vsel = jnp.where(pick_b, vb, va)

    @pl.when(pl.program_id(0) == 0)
    def _init():
        dsum_ref[...] = jnp.zeros_like(dsum_ref)

    dsum_ref[...] += jnp.sum(vsel)


def _assign(z, zsq, esq, embr_bf16):
    grid = (_ROWS // _R,)
    blocks_per_batch = 2048 // _R
    return pl.pallas_call(
        _assign_body,
        grid=grid,
        in_specs=[
            pl.BlockSpec((_R, 1), lambda i: (i, 0)),
            pl.BlockSpec((1, _N_EMB), lambda i: (0, 0)),
            pl.BlockSpec((1, _EMB_DIM, _R),
                         lambda i: (i // blocks_per_batch, 0,
                                    i % blocks_per_batch)),
            pl.BlockSpec((_N_EMB, _EMB_DIM), lambda i: (0, 0)),
        ],
        out_specs=[
            pl.BlockSpec((_R, 1), lambda i: (i, 0)),
            pl.BlockSpec((1, 1), lambda i: (0, 0)),
        ],
        out_shape=[
            jax.ShapeDtypeStruct((_ROWS, 1), jnp.int32),
            jax.ShapeDtypeStruct((1, 1), jnp.float32),
        ],
    )(zsq, esq, z, embr_bf16)


def _sc_gather_body(table_hbm, idx_hbm, out_hbm, idx_v, rows_v, sem):
    wid = lax.axis_index("s") * 2 + lax.axis_index("c")
    pltpu.sync_copy(idx_hbm.at[pl.ds(wid * _CH_PER_W, _CH_PER_W)], idx_v)
    copies = []
    for j in range(_CH_PER_W):
        copies.append(pltpu.async_copy(
            table_hbm.at[idx_v.at[j]],
            rows_v.at[pl.ds(j * _IDX_CHUNK, _IDX_CHUNK)],
            sem))
    for cp in copies:
        cp.wait()
    pltpu.sync_copy(rows_v, out_hbm.at[pl.ds(wid * _B_PER_W, _B_PER_W)])


def _sc_gather(emb_weight, idx_grid):
    return pl.kernel(
        _sc_gather_body,
        out_type=jax.ShapeDtypeStruct((_ROWS, _EMB_DIM), jnp.float32),
        mesh=plsc.VectorSubcoreMesh(core_axis_name="c", subcore_axis_name="s"),
        scratch_types=[
            pltpu.VMEM((_CH_PER_W, _IDX_CHUNK), jnp.int32),
            pltpu.VMEM((_B_PER_W, _EMB_DIM), jnp.float32),
            pltpu.SemaphoreType.DMA,
        ],
        compiler_params=pltpu.CompilerParams(use_tc_tiling_on_sc=False),
    )(emb_weight, idx_grid)


def kernel(z, emb_weight):
    zsq = jnp.sum(z ** 2, axis=1).reshape(_ROWS, 1)
    esq = jnp.sum(emb_weight ** 2, axis=1).reshape(1, _N_EMB)
    embr = (-2.0 * emb_weight).astype(jnp.bfloat16)

    idx2, dsum = _assign(z, zsq, esq, embr)
    encoding_indices = idx2.reshape(_ROWS)

    idx_grid = idx2.reshape(_NW * _CH_PER_W, _IDX_CHUNK)
    z_q_flat = _sc_gather(emb_weight, idx_grid)

    z_q_t = jnp.transpose(z_q_flat.reshape(8, 2048, _EMB_DIM), (0, 2, 1))
    m = dsum[0, 0] / (_ROWS * _EMB_DIM)
    loss = _BETA * m + m
    z_q_out = z + (z_q_t - z)
    return (z_q_out, loss, encoding_indices)
